# baseline (speedup 1.0000x reference)
import jax
import jax.numpy as jnp
from jax.experimental import pallas as pl
from jax.experimental.pallas import tpu as pltpu

N = 10000
D = 128
BM = 400


def _linear_kernel(x_ref, w_ref, out_ref):
    out_ref[...] = jnp.dot(
        x_ref[...], w_ref[...].T, preferred_element_type=jnp.float32
    )


def _agg_kernel(adj_ref, w_ref, b_ref, out_ref):
    acc = jnp.dot(adj_ref[...], w_ref[...], preferred_element_type=jnp.float32)
    out_ref[...] = jnp.maximum(acc + b_ref[...], 0.0)


def kernel(inputs, adj, W, b):
    w = pl.pallas_call(
        _linear_kernel,
        out_shape=jax.ShapeDtypeStruct((N, D), jnp.float32),
    )(inputs, W)

    b2 = b.reshape(1, D)

    grid = (N // BM,)
    out = pl.pallas_call(
        _agg_kernel,
        grid=grid,
        in_specs=[
            pl.BlockSpec((BM, N), lambda i: (i, 0)),
            pl.BlockSpec((N, D), lambda i: (0, 0)),
            pl.BlockSpec((1, D), lambda i: (0, 0)),
        ],
        out_specs=pl.BlockSpec((BM, D), lambda i: (i, 0)),
        out_shape=jax.ShapeDtypeStruct((N, D), jnp.float32),
        compiler_params=pltpu.CompilerParams(
            dimension_semantics=("parallel",),
        ),
    )(adj, w, b2)
    return out


# speedup vs baseline: 1.0297x; 1.0297x over previous
import jax
import jax.numpy as jnp
from jax.experimental import pallas as pl
from jax.experimental.pallas import tpu as pltpu

N = 10000
D = 128
BM = 400


def _gcn_kernel(x_ref, wt_ref, adj_ref, b_ref, out_ref, w_scratch):
    @pl.when(pl.program_id(0) == 0)
    def _():
        w_scratch[...] = jnp.dot(
            x_ref[...], wt_ref[...], preferred_element_type=jnp.float32
        )

    acc = jnp.dot(
        adj_ref[...], w_scratch[...], preferred_element_type=jnp.float32
    )
    out_ref[...] = jnp.maximum(acc + b_ref[...], 0.0)


def kernel(inputs, adj, W, b):
    b2 = b.reshape(1, D)
    wt = W.T

    grid = (N // BM,)
    out = pl.pallas_call(
        _gcn_kernel,
        grid=grid,
        in_specs=[
            pl.BlockSpec((N, D), lambda i: (0, 0)),
            pl.BlockSpec((D, D), lambda i: (0, 0)),
            pl.BlockSpec((BM, N), lambda i: (i, 0)),
            pl.BlockSpec((1, D), lambda i: (0, 0)),
        ],
        out_specs=pl.BlockSpec((BM, D), lambda i: (i, 0)),
        out_shape=jax.ShapeDtypeStruct((N, D), jnp.float32),
        scratch_shapes=[pltpu.VMEM((N, D), jnp.float32)],
        compiler_params=pltpu.CompilerParams(
            dimension_semantics=("arbitrary",),
        ),
    )(inputs, wt, adj, b2)
    return out
